# Initial kernel scaffold; baseline (speedup 1.0000x reference)
import jax
import jax.numpy as jnp
from jax.experimental import pallas as pl


def kernel(key_points, pos, edge_index, params):
    raise NotImplementedError("write your pallas kernel here")


# speedup vs baseline: 2.4073x; 2.4073x over previous
import functools

import jax
import jax.numpy as jnp
from jax import lax
from jax.experimental import pallas as pl
from jax.experimental.pallas import tpu as pltpu
from jax.experimental.pallas import tpu_sc as plsc

N = 10000
E = 160000
NP = 10240
F = 304
FV = F // 16
NC, NS = 2, 16
NW = NC * NS
RPW = NP // NW
CHUNK = 2000
GROUPS = CHUNK // 16
NCHUNK = E // CHUNK
FLUSH = 512
STAGE = 2560
CAP = E + FLUSH + 256
GB = 32
SENT = -3.0e38

@functools.lru_cache(maxsize=None)
def _mesh():
    return plsc.VectorSubcoreMesh(
        core_axis_name="c", subcore_axis_name="s", num_cores=NC, num_subcores=NS)


def _wid():
    return lax.axis_index("s") * NC + lax.axis_index("c")


def _compact_body(dst_hbm, src_hbm, csrc_hbm, cldst_hbm, counts_hbm,
                  dbuf, sbuf, stage_s, stage_d, cntb):
    w = _wid()
    lo = w * RPW
    hi = lo + RPW
    zero16 = jnp.zeros((16,), jnp.int32)

    def zed(i, _):
        stage_s[pl.ds(i * 16, 16)] = zero16
        stage_d[pl.ds(i * 16, 16)] = zero16
        return 0
    lax.fori_loop(0, STAGE // 16, zed, 0)

    def chunk_body(ci, carry):
        noff, flushed = carry
        pltpu.sync_copy(dst_hbm.at[pl.ds(pl.multiple_of(ci * CHUNK, 8), CHUNK)], dbuf)
        pltpu.sync_copy(src_hbm.at[pl.ds(pl.multiple_of(ci * CHUNK, 8), CHUNK)], sbuf)

        def grp(gi, off):
            d16 = dbuf[pl.ds(gi * 16, 16)]
            s16 = sbuf[pl.ds(gi * 16, 16)]
            msk = (d16 >= lo) & (d16 < hi)
            cs = plsc.cumsum(jnp.where(msk, 1, 0).astype(jnp.int32))
            pos = off + cs - 1
            plsc.store_scatter(stage_s, [pos], s16, mask=msk)
            plsc.store_scatter(stage_d, [pos], d16 - lo, mask=msk)
            return off + jnp.max(cs)
        noff = lax.fori_loop(0, GROUPS, grp, noff)

        nblk = noff // FLUSH

        def fl(i, _):
            pltpu.sync_copy(stage_s.at[pl.ds(i * FLUSH, FLUSH)],
                            csrc_hbm.at[pl.ds(pl.multiple_of(w * CAP + flushed + i * FLUSH, 8), FLUSH)])
            pltpu.sync_copy(stage_d.at[pl.ds(i * FLUSH, FLUSH)],
                            cldst_hbm.at[pl.ds(pl.multiple_of(w * CAP + flushed + i * FLUSH, 8), FLUSH)])
            return 0
        lax.fori_loop(0, nblk, fl, 0)

        base = nblk * FLUSH
        for j in range(FLUSH // 16):
            vs = stage_s[pl.ds(base + j * 16, 16)]
            vd = stage_d[pl.ds(base + j * 16, 16)]
            stage_s[pl.ds(j * 16, 16)] = vs
            stage_d[pl.ds(j * 16, 16)] = vd
        return (noff - base, flushed + base)

    noff, flushed = lax.fori_loop(
        0, NCHUNK, chunk_body, (jnp.int32(0), jnp.int32(0)))

    ntail = (noff + FLUSH - 1) // FLUSH

    def flt(i, _):
        pltpu.sync_copy(stage_s.at[pl.ds(i * FLUSH, FLUSH)],
                        csrc_hbm.at[pl.ds(pl.multiple_of(w * CAP + flushed + i * FLUSH, 8), FLUSH)])
        pltpu.sync_copy(stage_d.at[pl.ds(i * FLUSH, FLUSH)],
                        cldst_hbm.at[pl.ds(pl.multiple_of(w * CAP + flushed + i * FLUSH, 8), FLUSH)])
        return 0
    lax.fori_loop(0, ntail, flt, 0)

    cntb[...] = jnp.full((16,), flushed + noff, jnp.int32)
    pltpu.sync_copy(cntb, counts_hbm.at[pl.ds(pl.multiple_of(w * 16, 8), 16)])


@functools.lru_cache(maxsize=None)
def _compact():
    return pl.kernel(
        _compact_body,
        out_type=[jax.ShapeDtypeStruct((NW * CAP,), jnp.int32),
                  jax.ShapeDtypeStruct((NW * CAP,), jnp.int32),
                  jax.ShapeDtypeStruct((NW * 16,), jnp.int32)],
        mesh=_mesh(),
        compiler_params=pltpu.CompilerParams(needs_layout_passes=False),
        scratch_types=[pltpu.VMEM((CHUNK,), jnp.int32),
                       pltpu.VMEM((CHUNK,), jnp.int32),
                       pltpu.VMEM((STAGE,), jnp.int32),
                       pltpu.VMEM((STAGE,), jnp.int32),
                       pltpu.VMEM((16,), jnp.int32)],
    )


def _segmax_body(a_hbm, csrc_hbm, cldst_hbm, counts_hbm, m_hbm,
                 acc, rows, sidx, ldst, cntb, sem):
    w = _wid()
    lo = w * RPW
    sent = jnp.full((16,), SENT, jnp.float32)

    def ini(i, _):
        for j in range(FV):
            acc[i, pl.ds(j * 16, 16)] = sent
        return 0
    lax.fori_loop(0, RPW, ini, 0)

    pltpu.sync_copy(counts_hbm.at[pl.ds(pl.multiple_of(w * 16, 8), 16)], cntb)
    cnt = jnp.max(cntb[...])
    nb = (cnt + GB - 1) // GB
    iota = lax.iota(jnp.int32, 16)

    def batch(b, _):
        pltpu.sync_copy(csrc_hbm.at[pl.ds(pl.multiple_of(w * CAP + b * GB, 8), GB)], sidx)
        pltpu.sync_copy(cldst_hbm.at[pl.ds(pl.multiple_of(w * CAP + b * GB, 8), GB)], ldst)
        for j in range(GB // 16):
            v = sidx[pl.ds(j * 16, 16)]
            sidx[pl.ds(j * 16, 16)] = jnp.clip(v, 0, NP - 1)
        pltpu.async_copy(a_hbm.at[sidx], rows, sem).wait()
        mrem = jnp.minimum(GB, cnt - b * GB)

        def edge(k, _):
            g = k // 16
            lane = k - g * 16
            lv = ldst[pl.ds(g * 16, 16)]
            ld = jnp.max(jnp.where(iota == lane, lv, 0))
            for j in range(FV):
                av = acc[ld, pl.ds(j * 16, 16)]
                rv = rows[k, pl.ds(j * 16, 16)]
                acc[ld, pl.ds(j * 16, 16)] = jnp.maximum(av, rv)
            return 0
        lax.fori_loop(0, mrem, edge, 0)
        return 0
    lax.fori_loop(0, nb, batch, 0)

    pltpu.sync_copy(acc, m_hbm.at[pl.ds(pl.multiple_of(lo, 8), RPW)])


@functools.lru_cache(maxsize=None)
def _segmax():
    return pl.kernel(
        _segmax_body,
        out_type=jax.ShapeDtypeStruct((NP, F), jnp.float32),
        mesh=_mesh(),
        compiler_params=pltpu.CompilerParams(
            needs_layout_passes=False, use_tc_tiling_on_sc=False),
        scratch_types=[pltpu.VMEM((RPW, F), jnp.float32),
                       pltpu.VMEM((GB, F), jnp.float32),
                       pltpu.VMEM((GB,), jnp.int32),
                       pltpu.VMEM((GB,), jnp.int32),
                       pltpu.VMEM((16,), jnp.int32),
                       pltpu.SemaphoreType.DMA],
    )


def _inorm_relu(y):
    mean = jnp.mean(y, axis=-1, keepdims=True)
    var = jnp.mean((y - mean) ** 2, axis=-1, keepdims=True)
    return jax.nn.relu((y - mean) * lax.rsqrt(var + 1e-5))


def _dot(x, w):
    return jnp.dot(x, w, preferred_element_type=jnp.float32)


RB_INIT = 256
RB = 1024


def _init_body(kp_ref, w1, b1, w2, b2, w3, b3, w4, b4, out_ref):
    x = kp_ref[...].reshape(RB_INIT * 32, 8)
    h = _inorm_relu(_dot(x, w1[...]) + b1[...])
    h = _inorm_relu(_dot(h, w2[...]) + b2[...])
    h = _inorm_relu(_dot(h, w3[...]) + b3[...])
    h = _inorm_relu(_dot(h, w4[...]) + b4[...])
    s = jnp.max(h.reshape(RB_INIT, 32, 300), axis=1)
    out_ref[...] = jnp.concatenate(
        [s, jnp.zeros((RB_INIT, F - 300), jnp.float32)], axis=1)


def _tc_init(kp8, init_w):
    (w1, b1), (w2, b2), (w3, b3), (w4, b4) = init_w
    full = lambda a: pl.BlockSpec(a.shape, lambda i: (0,) * a.ndim)
    args = (w1, b1.reshape(1, -1), w2, b2.reshape(1, -1),
            w3, b3.reshape(1, -1), w4, b4.reshape(1, -1))
    return pl.pallas_call(
        _init_body,
        grid=(NP // RB_INIT,),
        in_specs=[pl.BlockSpec((RB_INIT, 32, 8), lambda i: (i, 0, 0))]
        + [full(a) for a in args],
        out_specs=pl.BlockSpec((RB_INIT, F), lambda i: (i, 0)),
        out_shape=jax.ShapeDtypeStruct((NP, F), jnp.float32),
    )(kp8, *args)


def _ad_body(s_ref, p_ref, wb, wha, wa, a_ref, d_ref):
    x = s_ref[...]
    pa = _dot(p_ref[...], wa[...])
    a_ref[...] = pa + _dot(x, wb[...])
    d_ref[...] = pa + _dot(x, wha[...])


def _tc_ad(s, pos8, wb, wha, wa):
    full = lambda a: pl.BlockSpec(a.shape, lambda i: (0,) * a.ndim)
    return pl.pallas_call(
        _ad_body,
        grid=(NP // RB,),
        in_specs=[pl.BlockSpec((RB, F), lambda i: (i, 0)),
                  pl.BlockSpec((RB, 8), lambda i: (i, 0)),
                  full(wb), full(wha), full(wa)],
        out_specs=[pl.BlockSpec((RB, F), lambda i: (i, 0)),
                   pl.BlockSpec((RB, F), lambda i: (i, 0))],
        out_shape=[jax.ShapeDtypeStruct((NP, F), jnp.float32),
                   jax.ShapeDtypeStruct((NP, F), jnp.float32)],
    )(s, pos8, wb, wha, wa)


def _update_body(m_ref, d_ref, s_ref, gw, gbc, out_ref):
    m = m_ref[...]
    z = jnp.where(m > -1e38, m - d_ref[...] + gbc[0:1], 0.0)
    out_ref[...] = s_ref[...] + _dot(z, gw[...]) + gbc[1:2]


def _tc_update(m, d, s, gw, gbc):
    full = lambda a: pl.BlockSpec(a.shape, lambda i: (0,) * a.ndim)
    return pl.pallas_call(
        _update_body,
        grid=(NP // RB,),
        in_specs=[pl.BlockSpec((RB, F), lambda i: (i, 0)),
                  pl.BlockSpec((RB, F), lambda i: (i, 0)),
                  pl.BlockSpec((RB, F), lambda i: (i, 0)),
                  full(gw), full(gbc)],
        out_specs=pl.BlockSpec((RB, F), lambda i: (i, 0)),
        out_shape=jax.ShapeDtypeStruct((NP, F), jnp.float32),
    )(m, d, s, gw, gbc)


def _heads_body(s_ref, cw1, cb1, cw2, cb2, lw1, lb1, lw2, lb2, lw3, lb3,
                cls_ref, reg_ref):
    t = s_ref[...][:, :300]
    c = _inorm_relu(_dot(t, cw1[...]) + cb1[...])
    cls_ref[...] = _inorm_relu(_dot(c, cw2[...]) + cb2[...])
    regs = []
    for h in range(4):
        r = _inorm_relu(_dot(t, lw1[h]) + lb1[h])
        r = _inorm_relu(_dot(r, lw2[h]) + lb2[h])
        r = _inorm_relu(_dot(r, lw3[h]) + lb3[h])
        regs.append(r)
    reg_ref[...] = jnp.concatenate(regs, axis=1)


def _tc_heads(s, cls_w, loc_w):
    (cw1, cb1), (cw2, cb2) = cls_w
    lw1 = jnp.stack([h[0][0] for h in loc_w])
    lb1 = jnp.stack([h[0][1] for h in loc_w]).reshape(4, 1, -1)
    lw2 = jnp.stack([h[1][0] for h in loc_w])
    lb2 = jnp.stack([h[1][1] for h in loc_w]).reshape(4, 1, -1)
    lw3 = jnp.stack([h[2][0] for h in loc_w])
    lb3 = jnp.stack([h[2][1] for h in loc_w]).reshape(4, 1, -1)
    full = lambda a: pl.BlockSpec(a.shape, lambda i: (0,) * a.ndim)
    args = (cw1, cb1.reshape(1, -1), cw2, cb2.reshape(1, -1),
            lw1, lb1, lw2, lb2, lw3, lb3)
    return pl.pallas_call(
        _heads_body,
        grid=(NP // RB,),
        in_specs=[pl.BlockSpec((RB, F), lambda i: (i, 0))]
        + [full(a) for a in args],
        out_specs=[pl.BlockSpec((RB, 4), lambda i: (i, 0)),
                   pl.BlockSpec((RB, 28), lambda i: (i, 0))],
        out_shape=[jax.ShapeDtypeStruct((NP, 4), jnp.float32),
                   jax.ShapeDtypeStruct((NP, 28), jnp.float32)],
    )(s, *args)


def _padf(w):
    return jnp.zeros((F, F), jnp.float32).at[:w.shape[0], :w.shape[1]].set(w)


def kernel(key_points, pos, edge_index, params):
    kp8 = jnp.pad(key_points, ((0, NP - N), (0, 0), (0, 8 - key_points.shape[2])))
    pos8 = jnp.pad(pos, ((0, NP - N), (0, 8 - pos.shape[1])))
    src = edge_index[0]
    dst = edge_index[1]

    layers = []
    for layer in params['gnn']:
        (h1, h1b), (h2, h2b) = layer['h']
        hw = h1 @ h2
        hb = h1b @ h2 + h2b
        (f1, f1b), (f2, f2b) = layer['f']
        fw = f1 @ f2
        fb = f1b @ f2 + f2b
        aa, bb = fw[:3], fw[3:]
        (g1, g1b), (g2, g2b) = layer['g']
        gw = g1 @ g2
        gb = g1b @ g2 + g2b
        c = fb - hb @ aa
        wa = jnp.zeros((8, F), jnp.float32).at[:3, :300].set(aa)
        gbc = jnp.zeros((2, F), jnp.float32).at[1, :300].set(gb).at[0, :300].set(c)
        layers.append((_padf(bb), _padf(hw @ aa), wa, _padf(gw), gbc))

    init_w = [(jnp.pad(params['init'][0][0], ((0, 4), (0, 0))), params['init'][0][1])] \
        + list(params['init'][1:])

    s = _tc_init(kp8, init_w)
    csrc, cldst, counts = _compact()(dst, src)
    for (wb, wha, wa, gw, gbc) in layers:
        a, d = _tc_ad(s, pos8, wb, wha, wa)
        m = _segmax()(a, csrc, cldst, counts)
        s = _tc_update(m, d, s, gw, gbc)
    cls, reg = _tc_heads(s, params['cls'], params['loc'])
    return (cls[:N][None], reg[:N][None])
